# baseline (speedup 1.0000x reference)
import functools

import jax
import jax.numpy as jnp
from jax import lax
from jax.experimental import pallas as pl
from jax.experimental.pallas import tpu as pltpu
from jax.experimental.pallas import tpu_sc as plsc

B = 16
LAT = 16
LEVELS = [25000, 6250, 1563, 391, 98]
SEQ = 9
OC = [16, 16, 16, 32]
IN_C = 3

_NC, _NS = 2, 16
_NW = _NC * _NS
_CH = 128
_GRAN = _NW * _CH
_BL = 64


def _rup(n, m):
    return (n + m - 1) // m * m


def _sc_mesh():
    return plsc.VectorSubcoreMesh(
        core_axis_name="c", subcore_axis_name="s",
        num_cores=_NC, num_subcores=_NS)


def _sc_gather_rows(table, idxp):
    N, D = table.shape
    (Mp,) = idxp.shape
    mw = Mp // _NW
    ch = 64 if D > 256 else _CH
    nch = mw // ch
    npair = nch // 2
    rem = nch - 2 * npair

    @functools.partial(
        pl.kernel,
        out_type=jax.ShapeDtypeStruct((Mp, D), jnp.float32),
        mesh=_sc_mesh(),
        scratch_types=[pltpu.VMEM((mw,), jnp.int32),
                       pltpu.VMEM((ch, D), jnp.float32),
                       pltpu.VMEM((ch, D), jnp.float32),
                       pltpu.SemaphoreType.DMA,
                       pltpu.SemaphoreType.DMA,
                       pltpu.SemaphoreType.DMA,
                       pltpu.SemaphoreType.DMA],
    )
    def k(tab_hbm, idx_hbm, out_hbm, idx_v, buf0, buf1, sg0, sg1, ss0, ss1):
        wid = lax.axis_index("s") * _NC + lax.axis_index("c")
        base = wid * mw
        pltpu.sync_copy(idx_hbm.at[pl.ds(base, mw)], idx_v)

        def gather(g, buf, sem):
            return pltpu.async_copy(
                tab_hbm.at[idx_v.at[pl.ds(g * ch, ch)]], buf, sem)

        def store(g, buf, sem):
            return pltpu.async_copy(
                buf, out_hbm.at[pl.ds(base + g * ch, ch)], sem)

        def wait_gather(g, buf, sem):
            pltpu.make_async_copy(
                tab_hbm.at[idx_v.at[pl.ds(g * ch, ch)]], buf, sem).wait()

        def wait_store(g, buf, sem):
            pltpu.make_async_copy(
                buf, out_hbm.at[pl.ds(base + g * ch, ch)], sem).wait()

        def body(h, carry):
            g0 = 2 * h
            g1 = g0 + 1

            @pl.when(h > 0)
            def _():
                wait_store(g0 - 2, buf0, ss0)

            gather(g0, buf0, sg0)

            @pl.when(h > 0)
            def _():
                wait_store(g1 - 2, buf1, ss1)

            gather(g1, buf1, sg1)
            wait_gather(g0, buf0, sg0)
            store(g0, buf0, ss0)
            wait_gather(g1, buf1, sg1)
            store(g1, buf1, ss1)
            return carry

        lax.fori_loop(0, npair, body, 0)
        if npair > 0:
            wait_store(2 * npair - 2, buf0, ss0)
            wait_store(2 * npair - 1, buf1, ss1)
        if rem:
            g = nch - 1
            gather(g, buf0, sg0).wait()
            store(g, buf0, ss0).wait()

    return k(table, idxp)


def _tc_pool_finish(R, val3, LP, D):
    nb = LP // _BL

    def body(r0_ref, r1_ref, r2_ref, v_ref, o_ref):
        v = v_ref[...]
        o_ref[...] = (v[:, 0:1] * r0_ref[...]
                      + v[:, 1:2] * r1_ref[...]
                      + v[:, 2:3] * r2_ref[...])

    return pl.pallas_call(
        body,
        grid=(nb,),
        in_specs=[pl.BlockSpec((_BL, D), lambda i, k=k: (k * nb + i, 0))
                  for k in range(3)]
        + [pl.BlockSpec((_BL, 3), lambda i: (i, 0))],
        out_specs=pl.BlockSpec((_BL, D), lambda i: (i, 0)),
        out_shape=jax.ShapeDtypeStruct((LP, D), jnp.float32),
    )(R, R, R, val3)


def _blockdiag(Wt, Cin, Cout):
    Ws = Wt.reshape(SEQ, Cin, Cout)
    eye = jnp.eye(B, dtype=jnp.float32)
    return jnp.einsum('ij,skl->sikjl', eye, Ws).reshape(SEQ, B * Cin, B * Cout)


def _tc_conv(G3, Wbd, bias, LP, Cin, Cout, elu):
    Din, Dout = B * Cin, B * Cout

    def body(g_ref, w_ref, b_ref, o_ref):
        acc = jnp.dot(g_ref[0], w_ref[0], preferred_element_type=jnp.float32)
        for s in range(1, SEQ):
            acc = acc + jnp.dot(g_ref[s], w_ref[s],
                                preferred_element_type=jnp.float32)
        y = acc + b_ref[...]
        if elu:
            y = jnp.where(y > 0, y, jnp.exp(jnp.minimum(y, 0.0)) - 1.0)
        o_ref[...] = y

    return pl.pallas_call(
        body,
        grid=(LP // _BL,),
        in_specs=[pl.BlockSpec((SEQ, _BL, Din), lambda i: (0, i, 0)),
                  pl.BlockSpec((SEQ, Din, Dout), lambda i: (0, 0, 0)),
                  pl.BlockSpec((1, Dout), lambda i: (0, 0))],
        out_specs=pl.BlockSpec((_BL, Dout), lambda i: (i, 0)),
        out_shape=jax.ShapeDtypeStruct((LP, Dout), jnp.float32),
    )(G3, Wbd, bias)


def _tc_qtables(X, Wbd, LP, Din, Dout):
    nb = LP // _BL

    def body(x_ref, w_ref, o_ref):
        o_ref[...] = jnp.dot(x_ref[...], w_ref[0],
                             preferred_element_type=jnp.float32)

    return pl.pallas_call(
        body,
        grid=(SEQ, nb),
        in_specs=[pl.BlockSpec((_BL, Din), lambda s, i: (i, 0)),
                  pl.BlockSpec((1, Din, Dout), lambda s, i: (s, 0, 0))],
        out_specs=pl.BlockSpec((_BL, Dout), lambda s, i: (s * nb + i, 0)),
        out_shape=jax.ShapeDtypeStruct((SEQ * LP, Dout), jnp.float32),
    )(X, Wbd)


def _tc_sum9(R, bias, LP, D):
    nb = LP // _BL

    def body(*refs):
        acc = refs[0][...]
        for s in range(1, SEQ):
            acc = acc + refs[s][...]
        refs[SEQ + 1][...] = acc + refs[SEQ][...]

    return pl.pallas_call(
        body,
        grid=(nb,),
        in_specs=[pl.BlockSpec((_BL, D), lambda i, s=s: (s * nb + i, 0))
                  for s in range(SEQ)]
        + [pl.BlockSpec((1, D), lambda i: (0, 0))],
        out_specs=pl.BlockSpec((_BL, D), lambda i: (i, 0)),
        out_shape=jax.ShapeDtypeStruct((LP, D), jnp.float32),
    )(*([R] * SEQ), bias)


def _pad_taps(taps, T, Lout, LP):
    Mp = _rup(T * LP, _GRAN)
    tT = jnp.zeros((T, LP), jnp.int32).at[:, :Lout].set(
        taps.T.astype(jnp.int32))
    return jnp.zeros((Mp,), jnp.int32).at[:T * LP].set(tT.reshape(-1))


def _level(X, col, val, sidx, Wt, bias, Lout, Cin, Cout, elu):
    LP = _rup(Lout, _BL)
    Din = B * Cin
    colp = _pad_taps(col.reshape(Lout, 3), 3, Lout, LP)
    Rf = _sc_gather_rows(X, colp)
    val3 = jnp.zeros((LP, 3), jnp.float32).at[:Lout].set(val.reshape(Lout, 3))
    P = _tc_pool_finish(Rf, val3, LP, Din)
    sidxp = _pad_taps(sidx, SEQ, Lout, LP)
    G3 = _sc_gather_rows(P, sidxp)[:SEQ * LP].reshape(SEQ, LP, Din)
    return _tc_conv(G3, _blockdiag(Wt, Cin, Cout), jnp.tile(bias, B)[None, :],
                    LP, Cin, Cout, elu)


def _heads_body(zmu_ref, zlv_ref, eps_ref,
                Wc1_ref, bc1_ref, gc1_ref, bec1_ref,
                Wc2_ref, bc2_ref, gc2_ref, bec2_ref,
                Wc3_ref, bc3_ref,
                Wr1_ref, br1_ref, gr1_ref, ber1_ref,
                Wr2_ref, br2_ref, gr2_ref, ber2_ref,
                Wr3_ref, br3_ref,
                z_ref, c_ref, r_ref):
    z = zmu_ref[...] + eps_ref[...] * jnp.exp(0.5 * zlv_ref[...])
    z_ref[...] = z

    def bn(v, g, b):
        m = v.mean(axis=0, keepdims=True)
        var = ((v - m) ** 2).mean(axis=0, keepdims=True)
        return (v - m) / jnp.sqrt(var + 1e-5) * g[None, :] + b[None, :]

    def lrelu(v):
        return jnp.where(v > 0, v, 0.2 * v)

    lane = jax.lax.broadcasted_iota(jnp.int32, (1, LAT), 1)
    ones8 = jnp.ones((LAT, 8), jnp.float32)
    zc_b = jnp.dot(jnp.where(lane == 0, z, 0.0), ones8,
                   preferred_element_type=jnp.float32)
    zr_b = jnp.dot(jnp.where(lane == 1, z, 0.0), ones8,
                   preferred_element_type=jnp.float32)

    c0 = zc_b * Wc1_ref[...] + bc1_ref[...][None, :]
    c = lrelu(bn(c0, gc1_ref[...], bec1_ref[...]))
    c1 = jnp.dot(c, Wc2_ref[...].T, preferred_element_type=jnp.float32) + bc2_ref[...][None, :]
    c = lrelu(bn(c1, gc2_ref[...], bec2_ref[...]))
    c2 = jnp.sum(c * Wc3_ref[...], axis=1, keepdims=True) + bc3_ref[...][None, :]
    c_ref[...] = jax.nn.sigmoid(c2)

    r0 = zr_b * Wr1_ref[...] + br1_ref[...][None, :]
    r = lrelu(bn(r0, gr1_ref[...], ber1_ref[...]))
    r1 = jnp.dot(r, Wr2_ref[...].T, preferred_element_type=jnp.float32) + br2_ref[...][None, :]
    r = lrelu(bn(r1, gr2_ref[...], ber2_ref[...]))
    r_ref[...] = jnp.sum(r * Wr3_ref[...], axis=1, keepdims=True) + br3_ref[...][None, :]


def kernel(x, idx, mu, log_var, eps, spiral0, spiral1, spiral2, spiral3, up_row0, up_col0, up_val0, up_row1, up_col1, up_val1, up_row2, up_col2, up_val2, up_row3, up_col3, up_val3, W_dec, b_dec, W_d0, b_d0, W_d1, b_d1, W_d2, b_d2, W_d3, b_d3, W_out, b_out, Wc1, bc1, gc1, bec1, Wc2, bc2, gc2, bec2, Wc3, bc3, Wr1, br1, gr1, ber1, Wr2, br2, gr2, ber2, Wr3, br3):
    z_mu = jnp.take(mu, idx, axis=0)
    z_lv = jnp.take(log_var, idx, axis=0)

    z, c, r = pl.pallas_call(
        _heads_body,
        out_shape=(
            jax.ShapeDtypeStruct((B, LAT), jnp.float32),
            jax.ShapeDtypeStruct((B, 1), jnp.float32),
            jax.ShapeDtypeStruct((B, 1), jnp.float32),
        ),
    )(z_mu, z_lv, eps,
      Wc1.T, bc1, gc1, bec1, Wc2, bc2, gc2, bec2, Wc3, bc3,
      Wr1.T, br1, gr1, ber1, Wr2, br2, gr2, ber2, Wr3, br3)

    h = (z @ W_dec.T + b_dec).reshape(B, LEVELS[4], OC[3])
    X = h.transpose(1, 0, 2).reshape(LEVELS[4], B * OC[3])

    X = _level(X, up_col3, up_val3, spiral3, W_d0.T, b_d0,
               LEVELS[3], OC[3], OC[3], True)
    X = _level(X, up_col2, up_val2, spiral2, W_d1.T, b_d1,
               LEVELS[2], OC[3], OC[2], True)
    X = _level(X, up_col1, up_val1, spiral1, W_d2.T, b_d2,
               LEVELS[1], OC[2], OC[1], True)
    X = _level(X, up_col0, up_val0, spiral0, W_d3.T, b_d3,
               LEVELS[0], OC[1], OC[0], True)

    LP0 = _rup(LEVELS[0], _BL)
    Wo = jnp.zeros((SEQ * OC[0], 8), jnp.float32).at[:, :IN_C].set(W_out.T)
    bo = jnp.zeros((8,), jnp.float32).at[:IN_C].set(b_out)
    Tq = _tc_qtables(X, _blockdiag(Wo, OC[0], 8), LP0, B * OC[0], B * 8)
    off = (jnp.arange(SEQ, dtype=jnp.int32) * LP0)[:, None]
    tT = (jnp.zeros((SEQ, LP0), jnp.int32)
          .at[:, :LEVELS[0]].set(spiral0.T.astype(jnp.int32)) + off)
    Mp = _rup(SEQ * LP0, _GRAN)
    idxp = jnp.zeros((Mp,), jnp.int32).at[:SEQ * LP0].set(tT.reshape(-1))
    Rf = _sc_gather_rows(Tq, idxp)
    Ys = _tc_sum9(Rf, jnp.tile(bo, B)[None, :], LP0, B * 8)
    out = Ys.reshape(LP0, B, 8)[:LEVELS[0], :, :IN_C].transpose(1, 0, 2)

    return (out, z_mu, z_lv, c, r)


# speedup vs baseline: 1.5778x; 1.5778x over previous
import functools

import jax
import jax.numpy as jnp
from jax import lax
from jax.experimental import pallas as pl
from jax.experimental.pallas import tpu as pltpu
from jax.experimental.pallas import tpu_sc as plsc

B = 16
LAT = 16
LEVELS = [25000, 6250, 1563, 391, 98]
SEQ = 9
OC = [16, 16, 16, 32]
IN_C = 3

_NC, _NS = 2, 16
_NW = _NC * _NS
_CH = 128
_GRAN = _NW * _CH
_BL = 64


def _rup(n, m):
    return (n + m - 1) // m * m


def _sc_mesh():
    return plsc.VectorSubcoreMesh(
        core_axis_name="c", subcore_axis_name="s",
        num_cores=_NC, num_subcores=_NS)


def _sc_gather_rows(table, idxp):
    N, D = table.shape
    (Mp,) = idxp.shape
    mw = Mp // _NW
    ch = 64 if D > 256 else _CH
    nch = mw // ch
    npair = nch // 2
    rem = nch - 2 * npair

    @functools.partial(
        pl.kernel,
        out_type=jax.ShapeDtypeStruct((Mp, D), jnp.float32),
        mesh=_sc_mesh(),
        scratch_types=[pltpu.VMEM((mw,), jnp.int32),
                       pltpu.VMEM((ch, D), jnp.float32),
                       pltpu.VMEM((ch, D), jnp.float32),
                       pltpu.SemaphoreType.DMA,
                       pltpu.SemaphoreType.DMA,
                       pltpu.SemaphoreType.DMA,
                       pltpu.SemaphoreType.DMA],
    )
    def k(tab_hbm, idx_hbm, out_hbm, idx_v, buf0, buf1, sg0, sg1, ss0, ss1):
        wid = lax.axis_index("s") * _NC + lax.axis_index("c")
        base = wid * mw
        pltpu.sync_copy(idx_hbm.at[pl.ds(base, mw)], idx_v)

        def gather(g, buf, sem):
            return pltpu.async_copy(
                tab_hbm.at[idx_v.at[pl.ds(g * ch, ch)]], buf, sem)

        def store(g, buf, sem):
            return pltpu.async_copy(
                buf, out_hbm.at[pl.ds(base + g * ch, ch)], sem)

        def wait_gather(g, buf, sem):
            pltpu.make_async_copy(
                tab_hbm.at[idx_v.at[pl.ds(g * ch, ch)]], buf, sem).wait()

        def wait_store(g, buf, sem):
            pltpu.make_async_copy(
                buf, out_hbm.at[pl.ds(base + g * ch, ch)], sem).wait()

        def body(h, carry):
            g0 = 2 * h
            g1 = g0 + 1

            @pl.when(h > 0)
            def _():
                wait_store(g0 - 2, buf0, ss0)

            gather(g0, buf0, sg0)

            @pl.when(h > 0)
            def _():
                wait_store(g1 - 2, buf1, ss1)

            gather(g1, buf1, sg1)
            wait_gather(g0, buf0, sg0)
            store(g0, buf0, ss0)
            wait_gather(g1, buf1, sg1)
            store(g1, buf1, ss1)
            return carry

        lax.fori_loop(0, npair, body, 0)
        if npair > 0:
            wait_store(2 * npair - 2, buf0, ss0)
            wait_store(2 * npair - 1, buf1, ss1)
        if rem:
            g = nch - 1
            gather(g, buf0, sg0).wait()
            store(g, buf0, ss0).wait()

    return k(table, idxp)


def _tc_pool_finish(R, val3, LP, D):
    nb = LP // _BL

    def body(r0_ref, r1_ref, r2_ref, v_ref, o_ref):
        v = v_ref[...]
        o_ref[...] = (v[:, 0:1] * r0_ref[...]
                      + v[:, 1:2] * r1_ref[...]
                      + v[:, 2:3] * r2_ref[...])

    return pl.pallas_call(
        body,
        grid=(nb,),
        in_specs=[pl.BlockSpec((_BL, D), lambda i, k=k: (k * nb + i, 0))
                  for k in range(3)]
        + [pl.BlockSpec((_BL, 3), lambda i: (i, 0))],
        out_specs=pl.BlockSpec((_BL, D), lambda i: (i, 0)),
        out_shape=jax.ShapeDtypeStruct((LP, D), jnp.float32),
    )(R, R, R, val3)


def _blockdiag(Wt, Cin, Cout):
    Ws = Wt.reshape(SEQ, Cin, Cout)
    eye = jnp.eye(B, dtype=jnp.float32)
    return jnp.einsum('ij,skl->sikjl', eye, Ws).reshape(SEQ, B * Cin, B * Cout)


def _tc_conv(G3, Wbd, bias, LP, Cin, Cout, elu):
    Din, Dout = B * Cin, B * Cout

    def body(g_ref, w_ref, b_ref, o_ref):
        acc = jnp.dot(g_ref[0], w_ref[0], preferred_element_type=jnp.float32)
        for s in range(1, SEQ):
            acc = acc + jnp.dot(g_ref[s], w_ref[s],
                                preferred_element_type=jnp.float32)
        y = acc + b_ref[...]
        if elu:
            y = jnp.where(y > 0, y, jnp.exp(jnp.minimum(y, 0.0)) - 1.0)
        o_ref[...] = y

    return pl.pallas_call(
        body,
        grid=(LP // _BL,),
        in_specs=[pl.BlockSpec((SEQ, _BL, Din), lambda i: (0, i, 0)),
                  pl.BlockSpec((SEQ, Din, Dout), lambda i: (0, 0, 0)),
                  pl.BlockSpec((1, Dout), lambda i: (0, 0))],
        out_specs=pl.BlockSpec((_BL, Dout), lambda i: (i, 0)),
        out_shape=jax.ShapeDtypeStruct((LP, Dout), jnp.float32),
    )(G3, Wbd, bias)


def _pad_taps(taps, T, Lout, LP):
    Mp = _rup(T * LP, _GRAN)
    tT = jnp.zeros((T, LP), jnp.int32).at[:, :Lout].set(
        taps.T.astype(jnp.int32))
    return jnp.zeros((Mp,), jnp.int32).at[:T * LP].set(tT.reshape(-1))


def _level(X, col, val, sidx, Wt, bias, Lout, Cin, Cout, elu):
    LP = _rup(Lout, _BL)
    Din = B * Cin
    colp = _pad_taps(col.reshape(Lout, 3), 3, Lout, LP)
    Rf = _sc_gather_rows(X, colp)
    val3 = jnp.zeros((LP, 3), jnp.float32).at[:Lout].set(val.reshape(Lout, 3))
    P = _tc_pool_finish(Rf, val3, LP, Din)
    sidxp = _pad_taps(sidx, SEQ, Lout, LP)
    G3 = _sc_gather_rows(P, sidxp)[:SEQ * LP].reshape(SEQ, LP, Din)
    return _tc_conv(G3, _blockdiag(Wt, Cin, Cout), jnp.tile(bias, B)[None, :],
                    LP, Cin, Cout, elu)


def _heads_body(zmu_ref, zlv_ref, eps_ref,
                Wc1_ref, bc1_ref, gc1_ref, bec1_ref,
                Wc2_ref, bc2_ref, gc2_ref, bec2_ref,
                Wc3_ref, bc3_ref,
                Wr1_ref, br1_ref, gr1_ref, ber1_ref,
                Wr2_ref, br2_ref, gr2_ref, ber2_ref,
                Wr3_ref, br3_ref,
                z_ref, c_ref, r_ref):
    z = zmu_ref[...] + eps_ref[...] * jnp.exp(0.5 * zlv_ref[...])
    z_ref[...] = z

    def bn(v, g, b):
        m = v.mean(axis=0, keepdims=True)
        var = ((v - m) ** 2).mean(axis=0, keepdims=True)
        return (v - m) / jnp.sqrt(var + 1e-5) * g[None, :] + b[None, :]

    def lrelu(v):
        return jnp.where(v > 0, v, 0.2 * v)

    lane = jax.lax.broadcasted_iota(jnp.int32, (1, LAT), 1)
    ones8 = jnp.ones((LAT, 8), jnp.float32)
    zc_b = jnp.dot(jnp.where(lane == 0, z, 0.0), ones8,
                   preferred_element_type=jnp.float32)
    zr_b = jnp.dot(jnp.where(lane == 1, z, 0.0), ones8,
                   preferred_element_type=jnp.float32)

    c0 = zc_b * Wc1_ref[...] + bc1_ref[...][None, :]
    c = lrelu(bn(c0, gc1_ref[...], bec1_ref[...]))
    c1 = jnp.dot(c, Wc2_ref[...].T, preferred_element_type=jnp.float32) + bc2_ref[...][None, :]
    c = lrelu(bn(c1, gc2_ref[...], bec2_ref[...]))
    c2 = jnp.sum(c * Wc3_ref[...], axis=1, keepdims=True) + bc3_ref[...][None, :]
    c_ref[...] = jax.nn.sigmoid(c2)

    r0 = zr_b * Wr1_ref[...] + br1_ref[...][None, :]
    r = lrelu(bn(r0, gr1_ref[...], ber1_ref[...]))
    r1 = jnp.dot(r, Wr2_ref[...].T, preferred_element_type=jnp.float32) + br2_ref[...][None, :]
    r = lrelu(bn(r1, gr2_ref[...], ber2_ref[...]))
    r_ref[...] = jnp.sum(r * Wr3_ref[...], axis=1, keepdims=True) + br3_ref[...][None, :]


def kernel(x, idx, mu, log_var, eps, spiral0, spiral1, spiral2, spiral3, up_row0, up_col0, up_val0, up_row1, up_col1, up_val1, up_row2, up_col2, up_val2, up_row3, up_col3, up_val3, W_dec, b_dec, W_d0, b_d0, W_d1, b_d1, W_d2, b_d2, W_d3, b_d3, W_out, b_out, Wc1, bc1, gc1, bec1, Wc2, bc2, gc2, bec2, Wc3, bc3, Wr1, br1, gr1, ber1, Wr2, br2, gr2, ber2, Wr3, br3):
    z_mu = jnp.take(mu, idx, axis=0)
    z_lv = jnp.take(log_var, idx, axis=0)

    z, c, r = pl.pallas_call(
        _heads_body,
        out_shape=(
            jax.ShapeDtypeStruct((B, LAT), jnp.float32),
            jax.ShapeDtypeStruct((B, 1), jnp.float32),
            jax.ShapeDtypeStruct((B, 1), jnp.float32),
        ),
    )(z_mu, z_lv, eps,
      Wc1.T, bc1, gc1, bec1, Wc2, bc2, gc2, bec2, Wc3, bc3,
      Wr1.T, br1, gr1, ber1, Wr2, br2, gr2, ber2, Wr3, br3)

    h = (z @ W_dec.T + b_dec).reshape(B, LEVELS[4], OC[3])
    X = h.transpose(1, 0, 2).reshape(LEVELS[4], B * OC[3])

    X = _level(X, up_col3, up_val3, spiral3, W_d0.T, b_d0,
               LEVELS[3], OC[3], OC[3], True)
    X = _level(X, up_col2, up_val2, spiral2, W_d1.T, b_d1,
               LEVELS[2], OC[3], OC[2], True)
    X = _level(X, up_col1, up_val1, spiral1, W_d2.T, b_d2,
               LEVELS[1], OC[2], OC[1], True)
    X = _level(X, up_col0, up_val0, spiral0, W_d3.T, b_d3,
               LEVELS[0], OC[1], OC[0], True)

    LP0 = _rup(LEVELS[0], _BL)
    Wo = jnp.zeros((SEQ * OC[0], 8), jnp.float32).at[:, :IN_C].set(W_out.T)
    bo = jnp.zeros((8,), jnp.float32).at[:IN_C].set(b_out)
    sidxp = _pad_taps(spiral0, SEQ, LEVELS[0], LP0)
    G3 = _sc_gather_rows(X, sidxp)[:SEQ * LP0].reshape(SEQ, LP0, B * OC[1])
    Ys = _tc_conv(G3, _blockdiag(Wo, OC[1], 8), jnp.tile(bo, B)[None, :],
                  LP0, OC[1], 8, False)
    out = Ys.reshape(LP0, B, 8)[:LEVELS[0], :, :IN_C].transpose(1, 0, 2)

    return (out, z_mu, z_lv, c, r)
